# Initial kernel scaffold; baseline (speedup 1.0000x reference)
import jax
import jax.numpy as jnp
from jax.experimental import pallas as pl


def kernel(x, table):
    raise NotImplementedError("write your pallas kernel here")


# speedup vs baseline: 5.2772x; 5.2772x over previous
import functools

import jax
import jax.numpy as jnp
from jax import lax
from jax.experimental import pallas as pl
from jax.experimental.pallas import tpu as pltpu
from jax.experimental.pallas import tpu_sc as plsc

_DIM = 8
_NC = 2
_NS = 16
_NW = _NC * _NS


@functools.lru_cache(maxsize=None)
def _build(n: int):
    assert n % _NW == 0
    per_w = n // _NW
    chunk = 3200
    assert per_w % chunk == 0
    n_chunks = per_w // chunk

    mesh = plsc.VectorSubcoreMesh(core_axis_name="c", subcore_axis_name="s")

    @functools.partial(
        pl.kernel,
        out_type=jax.ShapeDtypeStruct((n, _DIM), jnp.float32),
        mesh=mesh,
        scratch_types=[
            pltpu.VMEM((chunk,), jnp.int32),
            pltpu.VMEM((chunk, _DIM), jnp.float32),
            pltpu.SemaphoreType.DMA,
        ],
        compiler_params=pltpu.CompilerParams(use_tc_tiling_on_sc=False),
    )
    def gather_kernel(idx_hbm, table_hbm, out_hbm, idx_v, rows_v, sem):
        wid = lax.axis_index("s") * _NC + lax.axis_index("c")
        base = wid * per_w

        @pl.loop(0, n_chunks)
        def _(i):
            start = base + i * chunk
            pltpu.sync_copy(idx_hbm.at[pl.ds(start, chunk)], idx_v)
            pltpu.async_copy(table_hbm.at[idx_v], rows_v, sem).wait()
            pltpu.sync_copy(rows_v, out_hbm.at[pl.ds(start, chunk)])

    return gather_kernel


def kernel(x, table):
    flat = x.reshape(-1).astype(jnp.int32)
    out = _build(flat.shape[0])(flat, table)
    return out.reshape(x.shape + (_DIM,))


# speedup vs baseline: 5.3951x; 1.0223x over previous
import functools

import jax
import jax.numpy as jnp
from jax import lax
from jax.experimental import pallas as pl
from jax.experimental.pallas import tpu as pltpu
from jax.experimental.pallas import tpu_sc as plsc

_DIM = 8
_NC = 2
_NS = 16
_NW = _NC * _NS


@functools.lru_cache(maxsize=None)
def _build(n: int):
    assert n % _NW == 0
    per_w = n // _NW
    chunk = 3200
    assert per_w % chunk == 0
    n_chunks = per_w // chunk

    mesh = plsc.VectorSubcoreMesh(core_axis_name="c", subcore_axis_name="s")

    @functools.partial(
        pl.kernel,
        out_type=jax.ShapeDtypeStruct((n, _DIM), jnp.float32),
        mesh=mesh,
        scratch_types=[
            pltpu.VMEM((per_w,), jnp.int32),
            pltpu.VMEM((2, chunk, _DIM), jnp.float32),
            pltpu.SemaphoreType.DMA,
            pltpu.SemaphoreType.DMA,
            pltpu.SemaphoreType.DMA,
            pltpu.SemaphoreType.DMA,
        ],
        compiler_params=pltpu.CompilerParams(use_tc_tiling_on_sc=False),
    )
    def gather_kernel(idx_hbm, table_hbm, out_hbm, idx_v, rows_v,
                      sg0, sg1, so0, so1):
        wid = lax.axis_index("s") * _NC + lax.axis_index("c")
        base = wid * per_w
        sg = (sg0, sg1)
        so = (so0, so1)

        pltpu.sync_copy(idx_hbm.at[pl.ds(base, per_w)], idx_v)

        def gather(i):
            return pltpu.async_copy(
                table_hbm.at[idx_v.at[pl.ds(i * chunk, chunk)]],
                rows_v.at[i % 2], sg[i % 2])

        def writeback(i):
            return pltpu.async_copy(
                rows_v.at[i % 2],
                out_hbm.at[pl.ds(base + i * chunk, chunk)], so[i % 2])

        g = [None] * n_chunks
        w = [None] * n_chunks
        g[0] = gather(0)
        for i in range(1, n_chunks):
            b = i % 2
            if i >= 2:
                w[i - 2].wait()
            g[i] = gather(i)
            g[i - 1].wait()
            w[i - 1] = writeback(i - 1)
        g[n_chunks - 1].wait()
        if n_chunks >= 2:
            w[n_chunks - 2].wait()
        w[n_chunks - 1] = writeback(n_chunks - 1)
        w[n_chunks - 1].wait()

    return gather_kernel


def kernel(x, table):
    flat = x.reshape(-1).astype(jnp.int32)
    out = _build(flat.shape[0])(flat, table)
    return out.reshape(x.shape + (_DIM,))


# speedup vs baseline: 5.6330x; 1.0441x over previous
import functools

import jax
import jax.numpy as jnp
from jax import lax
from jax.experimental import pallas as pl
from jax.experimental.pallas import tpu as pltpu
from jax.experimental.pallas import tpu_sc as plsc

_DIM = 8
_NC = 2
_NS = 16
_NW = _NC * _NS


@functools.lru_cache(maxsize=None)
def _build(n: int):
    assert n % _NW == 0
    per_w = n // _NW
    chunk = 3200
    assert per_w % chunk == 0
    n_chunks = per_w // chunk

    mesh = plsc.VectorSubcoreMesh(core_axis_name="c", subcore_axis_name="s")

    @functools.partial(
        pl.kernel,
        out_type=jax.ShapeDtypeStruct((n, _DIM), jnp.float32),
        mesh=mesh,
        scratch_types=[
            pltpu.VMEM((per_w,), jnp.int32),
            pltpu.VMEM((2, chunk, _DIM), jnp.float32),
            pltpu.VMEM_SHARED((30000, _DIM), jnp.float32),
            pltpu.SemaphoreType.DMA,
            pltpu.SemaphoreType.DMA,
            pltpu.SemaphoreType.DMA,
            pltpu.SemaphoreType.DMA,
        ],
        compiler_params=pltpu.CompilerParams(use_tc_tiling_on_sc=False),
    )
    def gather_kernel(idx_hbm, table_hbm, out_hbm, idx_v, rows_v, table_s,
                      sg0, sg1, so0, so1):
        sid = lax.axis_index("s")
        wid = sid * _NC + lax.axis_index("c")
        base = wid * per_w
        sg = (sg0, sg1)
        so = (so0, so1)

        @pl.when(sid == 0)
        def _():
            pltpu.sync_copy(table_hbm, table_s)

        pltpu.sync_copy(idx_hbm.at[pl.ds(base, per_w)], idx_v)
        plsc.subcore_barrier()

        def gather(i):
            return pltpu.async_copy(
                table_s.at[idx_v.at[pl.ds(i * chunk, chunk)]],
                rows_v.at[i % 2], sg[i % 2])

        def writeback(i):
            return pltpu.async_copy(
                rows_v.at[i % 2],
                out_hbm.at[pl.ds(base + i * chunk, chunk)], so[i % 2])

        g = [None] * n_chunks
        w = [None] * n_chunks
        g[0] = gather(0)
        for i in range(1, n_chunks):
            b = i % 2
            if i >= 2:
                w[i - 2].wait()
            g[i] = gather(i)
            g[i - 1].wait()
            w[i - 1] = writeback(i - 1)
        g[n_chunks - 1].wait()
        if n_chunks >= 2:
            w[n_chunks - 2].wait()
        w[n_chunks - 1] = writeback(n_chunks - 1)
        w[n_chunks - 1].wait()

    return gather_kernel


def kernel(x, table):
    flat = x.reshape(-1).astype(jnp.int32)
    out = _build(flat.shape[0])(flat, table)
    return out.reshape(x.shape + (_DIM,))
